# baseline (speedup 1.0000x reference)
import functools

import jax
import jax.numpy as jnp
from jax import lax
from jax.experimental import pallas as pl
from jax.experimental.pallas import tpu as pltpu
from jax.experimental.pallas import tpu_sc as plsc

OUT_DIM = 1000
BATCH = 16384
CHANNEL = 512
HALF = CHANNEL // 2
LANES = 16
KVECS = HALF // LANES

NUM_WORKERS = 32
SW = 128
STRIPES_PER_WORKER = BATCH // (NUM_WORKERS * SW)
DEPTH = 2
PR = 2
NBANDS = 8
BAND = 128
BAND_SIZES = [BAND] * (NBANDS - 1) + [OUT_DIM - BAND * (NBANDS - 1)]


def _build():
    mesh = plsc.VectorSubcoreMesh(core_axis_name="c", subcore_axis_name="s")

    @functools.partial(
        pl.kernel,
        mesh=mesh,
        out_type=jax.ShapeDtypeStruct((OUT_DIM, BATCH), jnp.float32),
        scratch_types=[
            pltpu.VMEM((PR, CHANNEL), jnp.float32),
            pltpu.VMEM((PR, CHANNEL), jnp.float32),
            pltpu.VMEM((PR, HALF), jnp.int32),
            pltpu.VMEM((PR, HALF), jnp.int32),
            pltpu.VMEM((OUT_DIM, SW), jnp.float32),
            pltpu.SemaphoreType.DMA,
            pltpu.SemaphoreType.DMA,
            pltpu.SemaphoreType.DMA,
            pltpu.SemaphoreType.DMA,
            pltpu.SemaphoreType.DMA,
            pltpu.SemaphoreType.DMA,
            pltpu.SemaphoreType.DMA,
            pltpu.SemaphoreType.DMA,
            pltpu.SemaphoreType.DMA,
            pltpu.SemaphoreType.DMA,
        ],
        compiler_params=pltpu.CompilerParams(
            needs_layout_passes=False,
            disable_bounds_checks=True,
            disable_semaphore_checks=True,
        ),
    )
    def run(
        dec_hbm, cid_hbm, out_hbm,
        d0, d1, c0, c1, ob,
        si0, si1,
        sb0, sb1, sb2, sb3, sb4, sb5, sb6, sb7,
    ):
        wid = lax.axis_index("s") * 2 + lax.axis_index("c")
        dec_v = (d0, d1)
        cid_v = (c0, c1)
        sem_in = (si0, si1)
        sem_band = (sb0, sb1, sb2, sb3, sb4, sb5, sb6, sb7)
        zeros = jnp.zeros((LANES,), jnp.float32)

        def in_descs(row, t):
            return (
                pltpu.make_async_copy(
                    dec_hbm.at[pl.ds(row, PR)], dec_v[t], sem_in[t]
                ),
                pltpu.make_async_copy(
                    cid_hbm.at[pl.ds(row, PR)], cid_v[t], sem_in[t]
                ),
            )

        def start_in(row, t):
            a, b = in_descs(row, t)
            a.start()
            b.start()

        def wait_in(row, t):
            a, b = in_descs(row, t)
            a.wait()
            b.wait()

        def band_desc(k, col0):
            return pltpu.make_async_copy(
                ob.at[pl.ds(k * BAND, BAND_SIZES[k])],
                out_hbm.at[pl.ds(k * BAND, BAND_SIZES[k]), pl.ds(col0, SW)],
                sem_band[k],
            )

        def zero_band(k):
            def body(c, _):
                for j in range(SW // LANES):
                    ob[k * BAND + c, pl.ds(j * LANES, LANES)] = zeros
                return ()

            lax.fori_loop(0, BAND_SIZES[k], body, (), unroll=4)

        def scatter_pair(r, t):
            for rr in range(PR):
                colv = jnp.zeros((LANES,), jnp.int32) + (r + rr)
                for k in range(KVECS):
                    ids = cid_v[t][rr, pl.ds(k * LANES, LANES)]
                    a = dec_v[t][rr, pl.ds(k * LANES, LANES)]
                    b2 = dec_v[t][rr, pl.ds(HALF + k * LANES, LANES)]
                    plsc.addupdate_scatter(ob, [ids, colv], a + b2)

        for k in range(NBANDS):
            zero_band(k)
        col_base = pl.multiple_of(wid * STRIPES_PER_WORKER * SW, SW)
        for t in range(DEPTH):
            start_in(col_base + t * PR, t)

        def stripe_body(s, _):
            col0 = pl.multiple_of(col_base + s * SW, SW)

            STEP = DEPTH * PR
            def group(g, _):
                for t in range(DEPTH):
                    r = STEP * g + t * PR
                    wait_in(col0 + r, t)
                    scatter_pair(r, t)
                    start_in(col0 + r + STEP, t)
                return ()

            lax.fori_loop(0, SW // STEP - 2, group, ())

            base = SW - 2 * STEP
            for i in range(2 * DEPTH):
                t = i % DEPTH
                r = base + i * PR
                wait_in(col0 + r, t)
                scatter_pair(r, t)
                if i < DEPTH:
                    start_in(col0 + r + STEP, t)

            @pl.when(s < STRIPES_PER_WORKER - 1)
            def _():
                for t in range(DEPTH):
                    start_in(col0 + SW + t * PR, t)

            for k in range(NBANDS):
                band_desc(k, col0).start()
            for k in range(NBANDS):
                band_desc(k, col0).wait()
                zero_band(k)

            return ()

        lax.fori_loop(0, STRIPES_PER_WORKER, stripe_body, ())

    return run


_RUN = _build()


@jax.jit
def kernel(decoded, class_id):
    out_t = _RUN(decoded, class_id.astype(jnp.int32))
    return out_t.T


# speedup vs baseline: 1.1075x; 1.1075x over previous
import functools

import jax
import jax.numpy as jnp
from jax import lax
from jax.experimental import pallas as pl
from jax.experimental.pallas import tpu as pltpu
from jax.experimental.pallas import tpu_sc as plsc

OUT_DIM = 1000
BATCH = 16384
CHANNEL = 512
HALF = CHANNEL // 2
LANES = 16
KVECS = HALF // LANES

NUM_WORKERS = 32
SW = 128
STRIPES_PER_WORKER = BATCH // (NUM_WORKERS * SW)
DEPTH = 4
NBANDS = 8
BAND = 128
BAND_SIZES = [BAND] * (NBANDS - 1) + [OUT_DIM - BAND * (NBANDS - 1)]


def _build():
    mesh = plsc.VectorSubcoreMesh(core_axis_name="c", subcore_axis_name="s")

    @functools.partial(
        pl.kernel,
        mesh=mesh,
        out_type=jax.ShapeDtypeStruct((OUT_DIM, BATCH), jnp.float32),
        scratch_types=[
            pltpu.VMEM((1, CHANNEL), jnp.float32),
            pltpu.VMEM((1, CHANNEL), jnp.float32),
            pltpu.VMEM((1, CHANNEL), jnp.float32),
            pltpu.VMEM((1, CHANNEL), jnp.float32),
            pltpu.VMEM((1, HALF), jnp.int32),
            pltpu.VMEM((1, HALF), jnp.int32),
            pltpu.VMEM((1, HALF), jnp.int32),
            pltpu.VMEM((1, HALF), jnp.int32),
            pltpu.VMEM((OUT_DIM, SW), jnp.float32),
            pltpu.SemaphoreType.DMA,
            pltpu.SemaphoreType.DMA,
            pltpu.SemaphoreType.DMA,
            pltpu.SemaphoreType.DMA,
            pltpu.SemaphoreType.DMA,
            pltpu.SemaphoreType.DMA,
            pltpu.SemaphoreType.DMA,
            pltpu.SemaphoreType.DMA,
            pltpu.SemaphoreType.DMA,
            pltpu.SemaphoreType.DMA,
            pltpu.SemaphoreType.DMA,
            pltpu.SemaphoreType.DMA,
        ],
        compiler_params=pltpu.CompilerParams(
            needs_layout_passes=False,
            disable_bounds_checks=True,
            disable_semaphore_checks=True,
        ),
    )
    def run(
        dec_hbm, cid_hbm, out_hbm,
        d0, d1, d2, d3, c0, c1, c2, c3, ob,
        si0, si1, si2, si3,
        sb0, sb1, sb2, sb3, sb4, sb5, sb6, sb7,
    ):
        wid = lax.axis_index("s") * 2 + lax.axis_index("c")
        dec_v = (d0, d1, d2, d3)
        cid_v = (c0, c1, c2, c3)
        sem_in = (si0, si1, si2, si3)
        sem_band = (sb0, sb1, sb2, sb3, sb4, sb5, sb6, sb7)
        zeros = jnp.zeros((LANES,), jnp.float32)

        def in_descs(row, t):
            return (
                pltpu.make_async_copy(
                    dec_hbm.at[pl.ds(row, 1)], dec_v[t], sem_in[t]
                ),
                pltpu.make_async_copy(
                    cid_hbm.at[pl.ds(row, 1)], cid_v[t], sem_in[t]
                ),
            )

        def start_in(row, t):
            a, b = in_descs(row, t)
            a.start()
            b.start()

        def wait_in(row, t):
            a, b = in_descs(row, t)
            a.wait()
            b.wait()

        def band_desc(k, col0):
            return pltpu.make_async_copy(
                ob.at[pl.ds(k * BAND, BAND_SIZES[k])],
                out_hbm.at[pl.ds(k * BAND, BAND_SIZES[k]), pl.ds(col0, SW)],
                sem_band[k],
            )

        def zero_band(k):
            def body(c, _):
                for j in range(SW // LANES):
                    ob[k * BAND + c, pl.ds(j * LANES, LANES)] = zeros
                return ()

            lax.fori_loop(0, BAND_SIZES[k], body, (), unroll=4)

        def scatter_row(r, t):
            colv = jnp.zeros((LANES,), jnp.int32) + r
            for k in range(KVECS):
                ids = cid_v[t][0, pl.ds(k * LANES, LANES)]
                a = dec_v[t][0, pl.ds(k * LANES, LANES)]
                b2 = dec_v[t][0, pl.ds(HALF + k * LANES, LANES)]
                plsc.addupdate_scatter(ob, [ids, colv], a + b2)

        for k in range(NBANDS):
            zero_band(k)
        col_base = pl.multiple_of(wid * STRIPES_PER_WORKER * SW, SW)
        for t in range(DEPTH):
            start_in(col_base + t, t)

        def stripe_body(s, _):
            col0 = pl.multiple_of(col_base + s * SW, SW)

            def quad(g, _):
                for t in range(DEPTH):
                    r = DEPTH * g + t
                    wait_in(col0 + r, t)
                    scatter_row(r, t)
                    start_in(col0 + r + DEPTH, t)
                return ()

            lax.fori_loop(0, SW // DEPTH - 2, quad, ())

            base = SW - 2 * DEPTH
            for i in range(2 * DEPTH):
                t = i % DEPTH
                r = base + i
                wait_in(col0 + r, t)
                scatter_row(r, t)
                if i < DEPTH:
                    start_in(col0 + r + DEPTH, t)

            @pl.when(s < STRIPES_PER_WORKER - 1)
            def _():
                for t in range(DEPTH):
                    start_in(col0 + SW + t, t)

            for k in range(NBANDS):
                band_desc(k, col0).start()
            for k in range(NBANDS):
                band_desc(k, col0).wait()
                zero_band(k)

            return ()

        lax.fori_loop(0, STRIPES_PER_WORKER, stripe_body, ())

    return run


_RUN = _build()


@jax.jit
def kernel(decoded, class_id):
    out_t = _RUN(decoded, class_id.astype(jnp.int32))
    return out_t.T
